# baseline (speedup 1.0000x reference)
import jax
import jax.numpy as jnp
from jax import lax
from jax.experimental import pallas as pl
from jax.experimental.pallas import tpu as pltpu
from jax.experimental.pallas import tpu_sc as plsc

B = 16384
NUM_KCS = 1000
KC_PAD = 1024
EPSILON = 1e-08

NC = 2
NS = 16
NW = NC * NS
BPW = B // NW
CHUNK = 128
NCHUNK = BPW // CHUNK
STEPS = BPW // 16


def _sigmoid(x):
    return 1.0 / (1.0 + jnp.exp(-x))


def _bkt_body(kctab_hbm, omega_hbm, sigma_hbm,
              th0_hbm, th1_hbm, th2_hbm, th3_hbm, hp_hbm, obs_hbm,
              kc_hbm, pid_hbm, sid_hbm,
              hnew_hbm, pc_hbm,
              kctab_v, kc_v, obs_v, hp_v, pidx_v, sidx_v,
              om_v, sg_v, th0_v, th1_v, th2_v, th3_v, hn_v, pc_v,
              sem, gsem):
    wid = lax.axis_index("s") * NC + lax.axis_index("c")
    base = wid * BPW
    base2 = wid * (2 * BPW)

    idx_copies = [
        pltpu.async_copy(pid_hbm.at[pl.ds(base, BPW)], pidx_v, sem),
        pltpu.async_copy(sid_hbm.at[pl.ds(base, BPW)], sidx_v, sem),
    ]
    lin_copies = [
        pltpu.async_copy(kc_hbm.at[pl.ds(base, BPW)], kc_v, sem),
        pltpu.async_copy(obs_hbm.at[pl.ds(base, BPW)], obs_v, sem),
        pltpu.async_copy(hp_hbm.at[pl.ds(base2, 2 * BPW)], hp_v, sem),
        pltpu.async_copy(kctab_hbm, kctab_v, sem),
    ]
    for c in idx_copies:
        c.wait()

    gathers = []
    for j in range(NCHUNK):
        sl = pl.ds(j * CHUNK, CHUNK)
        gathers.append([
            pltpu.async_copy(omega_hbm.at[pidx_v.at[sl]], om_v.at[sl],
                             gsem.at[j]),
            pltpu.async_copy(sigma_hbm.at[pidx_v.at[sl]], sg_v.at[sl],
                             gsem.at[j]),
            pltpu.async_copy(th0_hbm.at[sidx_v.at[sl]], th0_v.at[sl],
                             gsem.at[j]),
            pltpu.async_copy(th1_hbm.at[sidx_v.at[sl]], th1_v.at[sl],
                             gsem.at[j]),
            pltpu.async_copy(th2_hbm.at[sidx_v.at[sl]], th2_v.at[sl],
                             gsem.at[j]),
            pltpu.async_copy(th3_hbm.at[sidx_v.at[sl]], th3_v.at[sl],
                             gsem.at[j]),
        ])
    for c in lin_copies:
        c.wait()

    def step(i, carry):
        s = pl.multiple_of(i * 16, 16)
        su = pl.multiple_of((i // 8) * 256 + (i % 8) * 16, 16)

        kidx = kc_v[pl.ds(s, 16)]
        pT_l = plsc.load_gather(kctab_v, [kidx])
        pF_l = plsc.load_gather(kctab_v, [kidx + KC_PAD])
        pG_l = plsc.load_gather(kctab_v, [kidx + 2 * KC_PAD])
        pS_l = plsc.load_gather(kctab_v, [kidx + 3 * KC_PAD])

        om = om_v[pl.ds(s, 16)]
        sg = sg_v[pl.ds(s, 16)]

        th_L = th0_v[pl.ds(s, 16)]
        th_nF = th1_v[pl.ds(s, 16)]
        th_G = th2_v[pl.ds(s, 16)]
        th_nS = th3_v[pl.ds(s, 16)]

        h_u = hp_v[pl.ds(su, 16)]
        h_m = hp_v[pl.ds(su + CHUNK, 16)]

        obs = obs_v[pl.ds(s, 16)]
        obs_b = obs != 0

        pT = _sigmoid(pT_l + th_L)
        pF = _sigmoid(pF_l - th_nF)
        pG = _sigmoid(pG_l + om + th_G)
        pS = _sigmoid(pS_l + sg - th_nS)

        p_m = jnp.where(obs_b, 1.0 - pS, pS)
        p_u = jnp.where(obs_b, pG, 1.0 - pG)

        a_u = p_u * h_u
        a_m = p_m * h_m
        new_m = (1.0 - pF) * a_m + pT * a_u
        new_u = pF * a_m + (1.0 - pT) * a_u
        inv_norm = 1.0 / (new_m + new_u + EPSILON)
        new_m = new_m * inv_norm
        new_u = new_u * inv_norm
        pc = (1.0 - pS) * new_m + pG * new_u

        hn_v[pl.ds(su, 16)] = new_u
        hn_v[pl.ds(su + CHUNK, 16)] = new_m
        pc_v[pl.ds(s, 16)] = pc
        return carry

    for j in range(NCHUNK):
        for g in gathers[j]:
            g.wait()
        for i in range(j * (STEPS // NCHUNK), (j + 1) * (STEPS // NCHUNK)):
            step(i, 0)

    pltpu.async_copy(hn_v, hnew_hbm.at[pl.ds(base2, 2 * BPW)], sem).wait()
    pltpu.async_copy(pc_v, pc_hbm.at[pl.ds(base, BPW)], sem).wait()


@jax.jit
def _bkt_sc(kctab, omega1d, sigma1d, th0, th1, th2, th3, hp_blk, observation,
            kc_ids, problem_ids, student_ids):
    mesh = plsc.VectorSubcoreMesh(core_axis_name="c", subcore_axis_name="s",
                                  num_cores=NC, num_subcores=NS)
    fn = pl.kernel(
        _bkt_body,
        out_type=(
            jax.ShapeDtypeStruct((2 * B,), jnp.float32),
            jax.ShapeDtypeStruct((B,), jnp.float32),
        ),
        mesh=mesh,
        scratch_types=[
            pltpu.VMEM((4 * KC_PAD,), jnp.float32),
            pltpu.VMEM((BPW,), jnp.int32),
            pltpu.VMEM((BPW,), jnp.int32),
            pltpu.VMEM((2 * BPW,), jnp.float32),
            pltpu.VMEM((BPW,), jnp.int32),
            pltpu.VMEM((BPW,), jnp.int32),
            pltpu.VMEM((BPW,), jnp.float32),
            pltpu.VMEM((BPW,), jnp.float32),
            pltpu.VMEM((BPW,), jnp.float32),
            pltpu.VMEM((BPW,), jnp.float32),
            pltpu.VMEM((BPW,), jnp.float32),
            pltpu.VMEM((BPW,), jnp.float32),
            pltpu.VMEM((2 * BPW,), jnp.float32),
            pltpu.VMEM((BPW,), jnp.float32),
            pltpu.SemaphoreType.DMA,
            pltpu.SemaphoreType.DMA((NCHUNK,)),
        ],
        compiler_params=pltpu.CompilerParams(needs_layout_passes=False,
                                             disable_bounds_checks=True),
        name="bkt_sc",
    )
    return fn(kctab, omega1d, sigma1d, th0, th1, th2, th3, hp_blk,
              observation, kc_ids, problem_ids, student_ids)


def kernel(h_prev, observation, kc_ids, problem_ids, student_ids,
           pT_logit, pF_logit, pG_logit, pS_logit, omega, sigma,
           student_ability):
    pad = KC_PAD - NUM_KCS
    kctab = jnp.concatenate([
        jnp.pad(pT_logit, (0, pad)),
        jnp.pad(pF_logit, (0, pad)),
        jnp.pad(pG_logit, (0, pad)),
        jnp.pad(pS_logit, (0, pad)),
    ])
    omega1d = omega.reshape(-1)
    sigma1d = sigma.reshape(-1)
    thT = student_ability.T
    hp_blk = h_prev.reshape(B // CHUNK, CHUNK, 2).transpose(0, 2, 1).reshape(-1)
    h_blk, p_correct = _bkt_sc(kctab, omega1d, sigma1d,
                               thT[0], thT[1], thT[2], thT[3],
                               hp_blk, observation,
                               kc_ids, problem_ids, student_ids)
    h_new = h_blk.reshape(B // CHUNK, 2, CHUNK).transpose(0, 2, 1).reshape(B, 2)
    return (h_new, p_correct)


# speedup vs baseline: 1.0177x; 1.0177x over previous
import jax
import jax.numpy as jnp
from jax import lax
from jax.experimental import pallas as pl
from jax.experimental.pallas import tpu as pltpu
from jax.experimental.pallas import tpu_sc as plsc

B = 16384
NUM_KCS = 1000
KC_PAD = 1024
EPSILON = 1e-08

NC = 2
NS = 16
NW = NC * NS
BPW = B // NW
CHUNK = 128
NCHUNK = BPW // CHUNK
STEPS = BPW // 16


def _sigmoid(x):
    return 1.0 / (1.0 + jnp.exp(-x))


def _bkt_body(kctab_hbm, omega_hbm, sigma_hbm,
              th0_hbm, th1_hbm, th2_hbm, th3_hbm, hp_hbm, obs_hbm,
              kc_hbm, pid_hbm, sid_hbm,
              hnew_hbm, pc_hbm,
              kctab_v, kc_v, obs_v, hp_v, pidx_v, sidx_v,
              om_v, sg_v, th0_v, th1_v, th2_v, th3_v, hn_v, pc_v,
              sem, gsem):
    wid = lax.axis_index("s") * NC + lax.axis_index("c")
    base = wid * BPW
    base2 = wid * (2 * BPW)

    idx_copies = [
        pltpu.async_copy(pid_hbm.at[pl.ds(base, BPW)], pidx_v, sem),
        pltpu.async_copy(sid_hbm.at[pl.ds(base, BPW)], sidx_v, sem),
    ]
    lin_copies = [
        pltpu.async_copy(kc_hbm.at[pl.ds(base, BPW)], kc_v, sem),
        pltpu.async_copy(obs_hbm.at[pl.ds(base, BPW)], obs_v, sem),
        pltpu.async_copy(hp_hbm.at[pl.ds(base2, 2 * BPW)], hp_v, sem),
        pltpu.async_copy(kctab_hbm, kctab_v, sem),
    ]
    for c in idx_copies:
        c.wait()

    gathers = []
    for j in range(NCHUNK):
        sl = pl.ds(j * CHUNK, CHUNK)
        gathers.append([
            pltpu.async_copy(omega_hbm.at[pidx_v.at[sl]], om_v.at[sl],
                             gsem.at[j]),
            pltpu.async_copy(sigma_hbm.at[pidx_v.at[sl]], sg_v.at[sl],
                             gsem.at[j]),
            pltpu.async_copy(th0_hbm.at[sidx_v.at[sl]], th0_v.at[sl],
                             gsem.at[j]),
            pltpu.async_copy(th1_hbm.at[sidx_v.at[sl]], th1_v.at[sl],
                             gsem.at[j]),
            pltpu.async_copy(th2_hbm.at[sidx_v.at[sl]], th2_v.at[sl],
                             gsem.at[j]),
            pltpu.async_copy(th3_hbm.at[sidx_v.at[sl]], th3_v.at[sl],
                             gsem.at[j]),
        ])
    for c in lin_copies:
        c.wait()

    def step(i, carry):
        s = pl.multiple_of(i * 16, 16)
        su = pl.multiple_of((i // 8) * 256 + (i % 8) * 16, 16)

        kidx = kc_v[pl.ds(s, 16)]
        pT_l = plsc.load_gather(kctab_v, [kidx])
        pF_l = plsc.load_gather(kctab_v, [kidx + KC_PAD])
        pG_l = plsc.load_gather(kctab_v, [kidx + 2 * KC_PAD])
        pS_l = plsc.load_gather(kctab_v, [kidx + 3 * KC_PAD])

        om = om_v[pl.ds(s, 16)]
        sg = sg_v[pl.ds(s, 16)]

        th_L = th0_v[pl.ds(s, 16)]
        th_nF = th1_v[pl.ds(s, 16)]
        th_G = th2_v[pl.ds(s, 16)]
        th_nS = th3_v[pl.ds(s, 16)]

        h_u = hp_v[pl.ds(su, 16)]
        h_m = hp_v[pl.ds(su + CHUNK, 16)]

        obs = obs_v[pl.ds(s, 16)]
        obs_b = obs != 0

        pT = _sigmoid(pT_l + th_L)
        pF = _sigmoid(pF_l - th_nF)
        pG = _sigmoid(pG_l + om + th_G)
        pS = _sigmoid(pS_l + sg - th_nS)

        p_m = jnp.where(obs_b, 1.0 - pS, pS)
        p_u = jnp.where(obs_b, pG, 1.0 - pG)

        a_u = p_u * h_u
        a_m = p_m * h_m
        new_m = (1.0 - pF) * a_m + pT * a_u
        new_u = pF * a_m + (1.0 - pT) * a_u
        inv_norm = 1.0 / (new_m + new_u + EPSILON)
        new_m = new_m * inv_norm
        new_u = new_u * inv_norm
        pc = (1.0 - pS) * new_m + pG * new_u

        hn_v[pl.ds(su, 16)] = new_u
        hn_v[pl.ds(su + CHUNK, 16)] = new_m
        pc_v[pl.ds(s, 16)] = pc
        return carry

    for j in range(NCHUNK):
        for g in gathers[j]:
            g.wait()
        lax.fori_loop(j * (STEPS // NCHUNK), (j + 1) * (STEPS // NCHUNK),
                      step, 0)

    pltpu.async_copy(hn_v, hnew_hbm.at[pl.ds(base2, 2 * BPW)], sem).wait()
    pltpu.async_copy(pc_v, pc_hbm.at[pl.ds(base, BPW)], sem).wait()


@jax.jit
def _bkt_sc(kctab, omega1d, sigma1d, th0, th1, th2, th3, hp_blk, observation,
            kc_ids, problem_ids, student_ids):
    mesh = plsc.VectorSubcoreMesh(core_axis_name="c", subcore_axis_name="s",
                                  num_cores=NC, num_subcores=NS)
    fn = pl.kernel(
        _bkt_body,
        out_type=(
            jax.ShapeDtypeStruct((2 * B,), jnp.float32),
            jax.ShapeDtypeStruct((B,), jnp.float32),
        ),
        mesh=mesh,
        scratch_types=[
            pltpu.VMEM((4 * KC_PAD,), jnp.float32),
            pltpu.VMEM((BPW,), jnp.int32),
            pltpu.VMEM((BPW,), jnp.int32),
            pltpu.VMEM((2 * BPW,), jnp.float32),
            pltpu.VMEM((BPW,), jnp.int32),
            pltpu.VMEM((BPW,), jnp.int32),
            pltpu.VMEM((BPW,), jnp.float32),
            pltpu.VMEM((BPW,), jnp.float32),
            pltpu.VMEM((BPW,), jnp.float32),
            pltpu.VMEM((BPW,), jnp.float32),
            pltpu.VMEM((BPW,), jnp.float32),
            pltpu.VMEM((BPW,), jnp.float32),
            pltpu.VMEM((2 * BPW,), jnp.float32),
            pltpu.VMEM((BPW,), jnp.float32),
            pltpu.SemaphoreType.DMA,
            pltpu.SemaphoreType.DMA((NCHUNK,)),
        ],
        compiler_params=pltpu.CompilerParams(needs_layout_passes=False,
                                             disable_bounds_checks=True),
        name="bkt_sc",
    )
    return fn(kctab, omega1d, sigma1d, th0, th1, th2, th3, hp_blk,
              observation, kc_ids, problem_ids, student_ids)


def kernel(h_prev, observation, kc_ids, problem_ids, student_ids,
           pT_logit, pF_logit, pG_logit, pS_logit, omega, sigma,
           student_ability):
    pad = KC_PAD - NUM_KCS
    kctab = jnp.concatenate([
        jnp.pad(pT_logit, (0, pad)),
        jnp.pad(pF_logit, (0, pad)),
        jnp.pad(pG_logit, (0, pad)),
        jnp.pad(pS_logit, (0, pad)),
    ])
    omega1d = omega.reshape(-1)
    sigma1d = sigma.reshape(-1)
    thT = student_ability.T
    hp_blk = h_prev.reshape(B // CHUNK, CHUNK, 2).transpose(0, 2, 1).reshape(-1)
    h_blk, p_correct = _bkt_sc(kctab, omega1d, sigma1d,
                               thT[0], thT[1], thT[2], thT[3],
                               hp_blk, observation,
                               kc_ids, problem_ids, student_ids)
    h_new = h_blk.reshape(B // CHUNK, 2, CHUNK).transpose(0, 2, 1).reshape(B, 2)
    return (h_new, p_correct)


# speedup vs baseline: 1.0406x; 1.0225x over previous
import jax
import jax.numpy as jnp
from jax import lax
from jax.experimental import pallas as pl
from jax.experimental.pallas import tpu as pltpu
from jax.experimental.pallas import tpu_sc as plsc

B = 16384
NUM_KCS = 1000
KC_PAD = 1024
EPSILON = 1e-08

NC = 2
NS = 16
NW = NC * NS
BPW = B // NW
CHUNK = 128
NCHUNK = BPW // CHUNK
STEPS = BPW // 16
SPC = STEPS // NCHUNK


def _sigmoid(x):
    return 1.0 / (1.0 + jnp.exp(-x))


def _logits_body(kctab_hbm, th0_hbm, th1_hbm, th2_hbm, th3_hbm,
                 kc_hbm, sid_hbm,
                 pT_hbm, pF_hbm, xg_hbm, xs_hbm,
                 kctab_v, kc_v, sidx_v,
                 th0_v, th1_v, th2_v, th3_v,
                 pT_v, pF_v, xg_v, xs_v, sem, gsem):
    wid = lax.axis_index("s") * NC + lax.axis_index("c")
    base = wid * BPW

    idx_copy = pltpu.async_copy(sid_hbm.at[pl.ds(base, BPW)], sidx_v, sem)
    lin_copies = [
        pltpu.async_copy(kc_hbm.at[pl.ds(base, BPW)], kc_v, sem),
        pltpu.async_copy(kctab_hbm, kctab_v, sem),
    ]
    idx_copy.wait()

    gathers = []
    for j in range(NCHUNK):
        sl = pl.ds(j * CHUNK, CHUNK)
        gathers.append([
            pltpu.async_copy(th0_hbm.at[sidx_v.at[sl]], th0_v.at[sl],
                             gsem.at[j]),
            pltpu.async_copy(th1_hbm.at[sidx_v.at[sl]], th1_v.at[sl],
                             gsem.at[j]),
            pltpu.async_copy(th2_hbm.at[sidx_v.at[sl]], th2_v.at[sl],
                             gsem.at[j]),
            pltpu.async_copy(th3_hbm.at[sidx_v.at[sl]], th3_v.at[sl],
                             gsem.at[j]),
        ])
    for c in lin_copies:
        c.wait()

    def step(i, carry):
        s = pl.multiple_of(i * 16, 16)
        kidx = kc_v[pl.ds(s, 16)]
        pT_l = plsc.load_gather(kctab_v, [kidx])
        pF_l = plsc.load_gather(kctab_v, [kidx + KC_PAD])
        pG_l = plsc.load_gather(kctab_v, [kidx + 2 * KC_PAD])
        pS_l = plsc.load_gather(kctab_v, [kidx + 3 * KC_PAD])
        th_L = th0_v[pl.ds(s, 16)]
        th_nF = th1_v[pl.ds(s, 16)]
        th_G = th2_v[pl.ds(s, 16)]
        th_nS = th3_v[pl.ds(s, 16)]
        pT_v[pl.ds(s, 16)] = _sigmoid(pT_l + th_L)
        pF_v[pl.ds(s, 16)] = _sigmoid(pF_l - th_nF)
        xg_v[pl.ds(s, 16)] = pG_l + th_G
        xs_v[pl.ds(s, 16)] = pS_l - th_nS
        return carry

    for j in range(NCHUNK):
        for g in gathers[j]:
            g.wait()
        lax.fori_loop(j * SPC, (j + 1) * SPC, step, 0)

    outs = [
        pltpu.async_copy(pT_v, pT_hbm.at[pl.ds(base, BPW)], sem),
        pltpu.async_copy(pF_v, pF_hbm.at[pl.ds(base, BPW)], sem),
        pltpu.async_copy(xg_v, xg_hbm.at[pl.ds(base, BPW)], sem),
        pltpu.async_copy(xs_v, xs_hbm.at[pl.ds(base, BPW)], sem),
    ]
    for o in outs:
        o.wait()


def _update_body(omega_hbm, sigma_hbm, hp_hbm, obs_hbm,
                 pT_hbm, pF_hbm, xg_hbm, xs_hbm, pid_hbm,
                 hnew_hbm, pc_hbm,
                 hp_v, obs_v, pidx_v, pT_v, pF_v, xg_v, xs_v,
                 om_v, sg_v, hn_v, pc_v, sem, gsem):
    wid = lax.axis_index("s") * NC + lax.axis_index("c")
    base = wid * BPW
    base2 = wid * (2 * BPW)

    idx_copy = pltpu.async_copy(pid_hbm.at[pl.ds(base, BPW)], pidx_v, sem)
    lin_copies = [
        pltpu.async_copy(hp_hbm.at[pl.ds(base2, 2 * BPW)], hp_v, sem),
        pltpu.async_copy(obs_hbm.at[pl.ds(base, BPW)], obs_v, sem),
        pltpu.async_copy(pT_hbm.at[pl.ds(base, BPW)], pT_v, sem),
        pltpu.async_copy(pF_hbm.at[pl.ds(base, BPW)], pF_v, sem),
        pltpu.async_copy(xg_hbm.at[pl.ds(base, BPW)], xg_v, sem),
        pltpu.async_copy(xs_hbm.at[pl.ds(base, BPW)], xs_v, sem),
    ]
    idx_copy.wait()

    gathers = []
    for j in range(NCHUNK):
        sl = pl.ds(j * CHUNK, CHUNK)
        gathers.append([
            pltpu.async_copy(omega_hbm.at[pidx_v.at[sl]], om_v.at[sl],
                             gsem.at[j]),
            pltpu.async_copy(sigma_hbm.at[pidx_v.at[sl]], sg_v.at[sl],
                             gsem.at[j]),
        ])
    for c in lin_copies:
        c.wait()

    def step(i, carry):
        s = pl.multiple_of(i * 16, 16)
        su = pl.multiple_of((i // 8) * 256 + (i % 8) * 16, 16)

        om = om_v[pl.ds(s, 16)]
        sg = sg_v[pl.ds(s, 16)]
        pT = pT_v[pl.ds(s, 16)]
        pF = pF_v[pl.ds(s, 16)]
        pG = _sigmoid(xg_v[pl.ds(s, 16)] + om)
        pS = _sigmoid(xs_v[pl.ds(s, 16)] + sg)

        h_u = hp_v[pl.ds(su, 16)]
        h_m = hp_v[pl.ds(su + CHUNK, 16)]
        obs_b = obs_v[pl.ds(s, 16)] != 0

        p_m = jnp.where(obs_b, 1.0 - pS, pS)
        p_u = jnp.where(obs_b, pG, 1.0 - pG)

        a_u = p_u * h_u
        a_m = p_m * h_m
        new_m = (1.0 - pF) * a_m + pT * a_u
        new_u = pF * a_m + (1.0 - pT) * a_u
        inv_norm = 1.0 / (new_m + new_u + EPSILON)
        new_m = new_m * inv_norm
        new_u = new_u * inv_norm
        pc = (1.0 - pS) * new_m + pG * new_u

        hn_v[pl.ds(su, 16)] = new_u
        hn_v[pl.ds(su + CHUNK, 16)] = new_m
        pc_v[pl.ds(s, 16)] = pc
        return carry

    for j in range(NCHUNK):
        for g in gathers[j]:
            g.wait()
        lax.fori_loop(j * SPC, (j + 1) * SPC, step, 0)

    pltpu.async_copy(hn_v, hnew_hbm.at[pl.ds(base2, 2 * BPW)], sem).wait()
    pltpu.async_copy(pc_v, pc_hbm.at[pl.ds(base, BPW)], sem).wait()


@jax.jit
def _bkt_sc(kctab, omega1d, sigma1d, th0, th1, th2, th3, hp_blk, observation,
            kc_ids, problem_ids, student_ids):
    mesh = plsc.VectorSubcoreMesh(core_axis_name="c", subcore_axis_name="s",
                                  num_cores=NC, num_subcores=NS)
    params = pltpu.CompilerParams(needs_layout_passes=False,
                                  disable_bounds_checks=True)
    out1d = jax.ShapeDtypeStruct((B,), jnp.float32)

    logits_fn = pl.kernel(
        _logits_body,
        out_type=(out1d, out1d, out1d, out1d),
        mesh=mesh,
        scratch_types=[
            pltpu.VMEM((4 * KC_PAD,), jnp.float32),
            pltpu.VMEM((BPW,), jnp.int32),
            pltpu.VMEM((BPW,), jnp.int32),
            pltpu.VMEM((BPW,), jnp.float32),
            pltpu.VMEM((BPW,), jnp.float32),
            pltpu.VMEM((BPW,), jnp.float32),
            pltpu.VMEM((BPW,), jnp.float32),
            pltpu.VMEM((BPW,), jnp.float32),
            pltpu.VMEM((BPW,), jnp.float32),
            pltpu.VMEM((BPW,), jnp.float32),
            pltpu.VMEM((BPW,), jnp.float32),
            pltpu.SemaphoreType.DMA,
            pltpu.SemaphoreType.DMA((NCHUNK,)),
        ],
        compiler_params=params,
        name="bkt_logits",
    )
    pT_a, pF_a, xg_a, xs_a = logits_fn(kctab, th0, th1, th2, th3,
                                       kc_ids, student_ids)

    update_fn = pl.kernel(
        _update_body,
        out_type=(
            jax.ShapeDtypeStruct((2 * B,), jnp.float32),
            out1d,
        ),
        mesh=mesh,
        scratch_types=[
            pltpu.VMEM((2 * BPW,), jnp.float32),
            pltpu.VMEM((BPW,), jnp.int32),
            pltpu.VMEM((BPW,), jnp.int32),
            pltpu.VMEM((BPW,), jnp.float32),
            pltpu.VMEM((BPW,), jnp.float32),
            pltpu.VMEM((BPW,), jnp.float32),
            pltpu.VMEM((BPW,), jnp.float32),
            pltpu.VMEM((BPW,), jnp.float32),
            pltpu.VMEM((BPW,), jnp.float32),
            pltpu.VMEM((2 * BPW,), jnp.float32),
            pltpu.VMEM((BPW,), jnp.float32),
            pltpu.SemaphoreType.DMA,
            pltpu.SemaphoreType.DMA((NCHUNK,)),
        ],
        compiler_params=params,
        name="bkt_update",
    )
    return update_fn(omega1d, sigma1d, hp_blk, observation,
                     pT_a, pF_a, xg_a, xs_a, problem_ids)


def kernel(h_prev, observation, kc_ids, problem_ids, student_ids,
           pT_logit, pF_logit, pG_logit, pS_logit, omega, sigma,
           student_ability):
    pad = KC_PAD - NUM_KCS
    kctab = jnp.concatenate([
        jnp.pad(pT_logit, (0, pad)),
        jnp.pad(pF_logit, (0, pad)),
        jnp.pad(pG_logit, (0, pad)),
        jnp.pad(pS_logit, (0, pad)),
    ])
    omega1d = omega.reshape(-1)
    sigma1d = sigma.reshape(-1)
    thT = student_ability.T
    hp_blk = h_prev.reshape(B // CHUNK, CHUNK, 2).transpose(0, 2, 1).reshape(-1)
    h_blk, p_correct = _bkt_sc(kctab, omega1d, sigma1d,
                               thT[0], thT[1], thT[2], thT[3],
                               hp_blk, observation,
                               kc_ids, problem_ids, student_ids)
    h_new = h_blk.reshape(B // CHUNK, 2, CHUNK).transpose(0, 2, 1).reshape(B, 2)
    return (h_new, p_correct)


# speedup vs baseline: 1.0688x; 1.0271x over previous
import jax
import jax.numpy as jnp
from jax import lax
from jax.experimental import pallas as pl
from jax.experimental.pallas import tpu as pltpu
from jax.experimental.pallas import tpu_sc as plsc

B = 16384
NUM_KCS = 1000
KC_PAD = 1024
EPSILON = 1e-08

NC = 2
NS = 16
NW = NC * NS
BPW = B // NW
CHUNK = 128
NCHUNK = BPW // CHUNK
STEPS = BPW // 16
SPC = STEPS // NCHUNK
NUM_STUDENTS = 100000
TH_MAIN = (NUM_STUDENTS // CHUNK) * CHUNK


def _sigmoid(x):
    return 1.0 / (1.0 + jnp.exp(-x))


def _logits_body(kctab_hbm, thblk_hbm,
                 kc_hbm, sid_hbm,
                 pT_hbm, pF_hbm, xg_hbm, xs_hbm,
                 kctab_v, kc_v, sidx_v,
                 ti0_v, ti1_v, ti2_v, ti3_v,
                 th0_v, th1_v, th2_v, th3_v,
                 pT_v, pF_v, xg_v, xs_v, sem, gsem):
    wid = lax.axis_index("s") * NC + lax.axis_index("c")
    base = wid * BPW

    idx_copy = pltpu.async_copy(sid_hbm.at[pl.ds(base, BPW)], sidx_v, sem)
    lin_copies = [
        pltpu.async_copy(kc_hbm.at[pl.ds(base, BPW)], kc_v, sem),
        pltpu.async_copy(kctab_hbm, kctab_v, sem),
    ]
    idx_copy.wait()

    for i in range(STEPS):
        s = pl.multiple_of(i * 16, 16)
        sid = sidx_v[pl.ds(s, 16)]
        low = sid & 127
        main0 = (sid & ~127) * 4 + low
        in_tail = sid >= TH_MAIN
        tail0 = TH_MAIN * 4 + (sid - TH_MAIN)
        ti0_v[pl.ds(s, 16)] = jnp.where(in_tail, tail0, main0)
        ti1_v[pl.ds(s, 16)] = jnp.where(in_tail, tail0 + 32, main0 + 128)
        ti2_v[pl.ds(s, 16)] = jnp.where(in_tail, tail0 + 64, main0 + 256)
        ti3_v[pl.ds(s, 16)] = jnp.where(in_tail, tail0 + 96, main0 + 384)

    gathers = []
    for j in range(NCHUNK):
        sl = pl.ds(j * CHUNK, CHUNK)
        gathers.append([
            pltpu.async_copy(thblk_hbm.at[ti0_v.at[sl]], th0_v.at[sl],
                             gsem.at[j]),
            pltpu.async_copy(thblk_hbm.at[ti1_v.at[sl]], th1_v.at[sl],
                             gsem.at[j]),
            pltpu.async_copy(thblk_hbm.at[ti2_v.at[sl]], th2_v.at[sl],
                             gsem.at[j]),
            pltpu.async_copy(thblk_hbm.at[ti3_v.at[sl]], th3_v.at[sl],
                             gsem.at[j]),
        ])
    for c in lin_copies:
        c.wait()

    def step(i, carry):
        s = pl.multiple_of(i * 16, 16)
        kidx = kc_v[pl.ds(s, 16)]
        pT_l = plsc.load_gather(kctab_v, [kidx])
        pF_l = plsc.load_gather(kctab_v, [kidx + KC_PAD])
        pG_l = plsc.load_gather(kctab_v, [kidx + 2 * KC_PAD])
        pS_l = plsc.load_gather(kctab_v, [kidx + 3 * KC_PAD])
        th_L = th0_v[pl.ds(s, 16)]
        th_nF = th1_v[pl.ds(s, 16)]
        th_G = th2_v[pl.ds(s, 16)]
        th_nS = th3_v[pl.ds(s, 16)]
        pT_v[pl.ds(s, 16)] = _sigmoid(pT_l + th_L)
        pF_v[pl.ds(s, 16)] = _sigmoid(pF_l - th_nF)
        xg_v[pl.ds(s, 16)] = pG_l + th_G
        xs_v[pl.ds(s, 16)] = pS_l - th_nS
        return carry

    for j in range(NCHUNK):
        for g in gathers[j]:
            g.wait()
        lax.fori_loop(j * SPC, (j + 1) * SPC, step, 0)

    outs = [
        pltpu.async_copy(pT_v, pT_hbm.at[pl.ds(base, BPW)], sem),
        pltpu.async_copy(pF_v, pF_hbm.at[pl.ds(base, BPW)], sem),
        pltpu.async_copy(xg_v, xg_hbm.at[pl.ds(base, BPW)], sem),
        pltpu.async_copy(xs_v, xs_hbm.at[pl.ds(base, BPW)], sem),
    ]
    for o in outs:
        o.wait()


def _update_body(omega_hbm, sigma_hbm, hp_hbm, obs_hbm,
                 pT_hbm, pF_hbm, xg_hbm, xs_hbm, pid_hbm,
                 hnew_hbm, pc_hbm,
                 hp_v, obs_v, pidx_v, pT_v, pF_v, xg_v, xs_v,
                 om_v, sg_v, hn_v, pc_v, sem, gsem):
    wid = lax.axis_index("s") * NC + lax.axis_index("c")
    base = wid * BPW
    base2 = wid * (2 * BPW)

    idx_copy = pltpu.async_copy(pid_hbm.at[pl.ds(base, BPW)], pidx_v, sem)
    lin_copies = [
        pltpu.async_copy(hp_hbm.at[pl.ds(base2, 2 * BPW)], hp_v, sem),
        pltpu.async_copy(obs_hbm.at[pl.ds(base, BPW)], obs_v, sem),
        pltpu.async_copy(pT_hbm.at[pl.ds(base, BPW)], pT_v, sem),
        pltpu.async_copy(pF_hbm.at[pl.ds(base, BPW)], pF_v, sem),
        pltpu.async_copy(xg_hbm.at[pl.ds(base, BPW)], xg_v, sem),
        pltpu.async_copy(xs_hbm.at[pl.ds(base, BPW)], xs_v, sem),
    ]
    idx_copy.wait()

    gathers = []
    for j in range(NCHUNK):
        sl = pl.ds(j * CHUNK, CHUNK)
        gathers.append([
            pltpu.async_copy(omega_hbm.at[pidx_v.at[sl]], om_v.at[sl],
                             gsem.at[j]),
            pltpu.async_copy(sigma_hbm.at[pidx_v.at[sl]], sg_v.at[sl],
                             gsem.at[j]),
        ])
    for c in lin_copies:
        c.wait()

    def step(i, carry):
        s = pl.multiple_of(i * 16, 16)
        su = pl.multiple_of((i // 8) * 256 + (i % 8) * 16, 16)

        om = om_v[pl.ds(s, 16)]
        sg = sg_v[pl.ds(s, 16)]
        pT = pT_v[pl.ds(s, 16)]
        pF = pF_v[pl.ds(s, 16)]
        pG = _sigmoid(xg_v[pl.ds(s, 16)] + om)
        pS = _sigmoid(xs_v[pl.ds(s, 16)] + sg)

        h_u = hp_v[pl.ds(su, 16)]
        h_m = hp_v[pl.ds(su + CHUNK, 16)]
        obs_b = obs_v[pl.ds(s, 16)] != 0

        p_m = jnp.where(obs_b, 1.0 - pS, pS)
        p_u = jnp.where(obs_b, pG, 1.0 - pG)

        a_u = p_u * h_u
        a_m = p_m * h_m
        new_m = (1.0 - pF) * a_m + pT * a_u
        new_u = pF * a_m + (1.0 - pT) * a_u
        inv_norm = 1.0 / (new_m + new_u + EPSILON)
        new_m = new_m * inv_norm
        new_u = new_u * inv_norm
        pc = (1.0 - pS) * new_m + pG * new_u

        hn_v[pl.ds(su, 16)] = new_u
        hn_v[pl.ds(su + CHUNK, 16)] = new_m
        pc_v[pl.ds(s, 16)] = pc
        return carry

    for j in range(NCHUNK):
        for g in gathers[j]:
            g.wait()
        lax.fori_loop(j * SPC, (j + 1) * SPC, step, 0)

    pltpu.async_copy(hn_v, hnew_hbm.at[pl.ds(base2, 2 * BPW)], sem).wait()
    pltpu.async_copy(pc_v, pc_hbm.at[pl.ds(base, BPW)], sem).wait()


@jax.jit
def _bkt_sc(kctab, omega1d, sigma1d, thblk, hp_blk, observation,
            kc_ids, problem_ids, student_ids):
    mesh = plsc.VectorSubcoreMesh(core_axis_name="c", subcore_axis_name="s",
                                  num_cores=NC, num_subcores=NS)
    params = pltpu.CompilerParams(needs_layout_passes=False,
                                  disable_bounds_checks=True)
    out1d = jax.ShapeDtypeStruct((B,), jnp.float32)

    logits_fn = pl.kernel(
        _logits_body,
        out_type=(out1d, out1d, out1d, out1d),
        mesh=mesh,
        scratch_types=[
            pltpu.VMEM((4 * KC_PAD,), jnp.float32),
            pltpu.VMEM((BPW,), jnp.int32),
            pltpu.VMEM((BPW,), jnp.int32),
            pltpu.VMEM((BPW,), jnp.int32),
            pltpu.VMEM((BPW,), jnp.int32),
            pltpu.VMEM((BPW,), jnp.int32),
            pltpu.VMEM((BPW,), jnp.int32),
            pltpu.VMEM((BPW,), jnp.float32),
            pltpu.VMEM((BPW,), jnp.float32),
            pltpu.VMEM((BPW,), jnp.float32),
            pltpu.VMEM((BPW,), jnp.float32),
            pltpu.VMEM((BPW,), jnp.float32),
            pltpu.VMEM((BPW,), jnp.float32),
            pltpu.VMEM((BPW,), jnp.float32),
            pltpu.VMEM((BPW,), jnp.float32),
            pltpu.SemaphoreType.DMA,
            pltpu.SemaphoreType.DMA((NCHUNK,)),
        ],
        compiler_params=params,
        name="bkt_logits",
    )
    pT_a, pF_a, xg_a, xs_a = logits_fn(kctab, thblk, kc_ids, student_ids)

    update_fn = pl.kernel(
        _update_body,
        out_type=(
            jax.ShapeDtypeStruct((2 * B,), jnp.float32),
            out1d,
        ),
        mesh=mesh,
        scratch_types=[
            pltpu.VMEM((2 * BPW,), jnp.float32),
            pltpu.VMEM((BPW,), jnp.int32),
            pltpu.VMEM((BPW,), jnp.int32),
            pltpu.VMEM((BPW,), jnp.float32),
            pltpu.VMEM((BPW,), jnp.float32),
            pltpu.VMEM((BPW,), jnp.float32),
            pltpu.VMEM((BPW,), jnp.float32),
            pltpu.VMEM((BPW,), jnp.float32),
            pltpu.VMEM((BPW,), jnp.float32),
            pltpu.VMEM((2 * BPW,), jnp.float32),
            pltpu.VMEM((BPW,), jnp.float32),
            pltpu.SemaphoreType.DMA,
            pltpu.SemaphoreType.DMA((NCHUNK,)),
        ],
        compiler_params=params,
        name="bkt_update",
    )
    return update_fn(omega1d, sigma1d, hp_blk, observation,
                     pT_a, pF_a, xg_a, xs_a, problem_ids)


def kernel(h_prev, observation, kc_ids, problem_ids, student_ids,
           pT_logit, pF_logit, pG_logit, pS_logit, omega, sigma,
           student_ability):
    pad = KC_PAD - NUM_KCS
    kctab = jnp.concatenate([
        jnp.pad(pT_logit, (0, pad)),
        jnp.pad(pF_logit, (0, pad)),
        jnp.pad(pG_logit, (0, pad)),
        jnp.pad(pS_logit, (0, pad)),
    ])
    omega1d = omega.reshape(-1)
    sigma1d = sigma.reshape(-1)
    thblk = jnp.concatenate([
        student_ability[:TH_MAIN]
        .reshape(TH_MAIN // CHUNK, CHUNK, 4).transpose(0, 2, 1).reshape(-1),
        student_ability[TH_MAIN:].T.reshape(-1),
    ])
    hp_blk = h_prev.reshape(B // CHUNK, CHUNK, 2).transpose(0, 2, 1).reshape(-1)
    h_blk, p_correct = _bkt_sc(kctab, omega1d, sigma1d, thblk,
                               hp_blk, observation,
                               kc_ids, problem_ids, student_ids)
    h_new = h_blk.reshape(B // CHUNK, 2, CHUNK).transpose(0, 2, 1).reshape(B, 2)
    return (h_new, p_correct)


# speedup vs baseline: 1.0846x; 1.0147x over previous
import jax
import jax.numpy as jnp
from jax import lax
from jax.experimental import pallas as pl
from jax.experimental.pallas import tpu as pltpu
from jax.experimental.pallas import tpu_sc as plsc

B = 16384
NUM_KCS = 1000
KC_PAD = 1024
EPSILON = 1e-08

NC = 2
NS = 16
NW = NC * NS
BPW = B // NW
CHUNK = 128
NCHUNK = BPW // CHUNK
STEPS = BPW // 16
SPC = STEPS // NCHUNK
NUM_STUDENTS = 100000
TH_MAIN = (NUM_STUDENTS // CHUNK) * CHUNK


def _sigmoid(x):
    return 1.0 / (1.0 + jnp.exp(-x))


def _logits_body(kctab_hbm, thblk_hbm,
                 kc_hbm, sid_hbm,
                 pT_hbm, pF_hbm, xg_hbm, xs_hbm,
                 kctab_v, kc_v, sidx_v,
                 ti0_v, ti1_v, ti2_v, ti3_v,
                 th0_v, th1_v, th2_v, th3_v,
                 pT_v, pF_v, xg_v, xs_v, sem, gsem):
    wid = lax.axis_index("s") * NC + lax.axis_index("c")
    base = wid * BPW

    idx_copy = pltpu.async_copy(sid_hbm.at[pl.ds(base, BPW)], sidx_v, sem)
    lin_copies = [
        pltpu.async_copy(kc_hbm.at[pl.ds(base, BPW)], kc_v, sem),
        pltpu.async_copy(kctab_hbm, kctab_v, sem),
    ]
    idx_copy.wait()

    for i in range(STEPS):
        s = pl.multiple_of(i * 16, 16)
        sid = sidx_v[pl.ds(s, 16)]
        low = sid & 127
        main0 = (sid & ~127) * 4 + low
        in_tail = sid >= TH_MAIN
        tail0 = TH_MAIN * 4 + (sid - TH_MAIN)
        ti0_v[pl.ds(s, 16)] = jnp.where(in_tail, tail0, main0)
        ti1_v[pl.ds(s, 16)] = jnp.where(in_tail, tail0 + 32, main0 + 128)
        ti2_v[pl.ds(s, 16)] = jnp.where(in_tail, tail0 + 64, main0 + 256)
        ti3_v[pl.ds(s, 16)] = jnp.where(in_tail, tail0 + 96, main0 + 384)

    gathers = []
    for j in range(NCHUNK):
        sl = pl.ds(j * CHUNK, CHUNK)
        gathers.append([
            pltpu.async_copy(thblk_hbm.at[ti0_v.at[sl]], th0_v.at[sl],
                             gsem.at[j]),
            pltpu.async_copy(thblk_hbm.at[ti1_v.at[sl]], th1_v.at[sl],
                             gsem.at[j]),
            pltpu.async_copy(thblk_hbm.at[ti2_v.at[sl]], th2_v.at[sl],
                             gsem.at[j]),
            pltpu.async_copy(thblk_hbm.at[ti3_v.at[sl]], th3_v.at[sl],
                             gsem.at[j]),
        ])
    for c in lin_copies:
        c.wait()

    def step(i, carry):
        s = pl.multiple_of(i * 16, 16)
        kidx = kc_v[pl.ds(s, 16)]
        pT_l = plsc.load_gather(kctab_v, [kidx])
        pF_l = plsc.load_gather(kctab_v, [kidx + KC_PAD])
        pG_l = plsc.load_gather(kctab_v, [kidx + 2 * KC_PAD])
        pS_l = plsc.load_gather(kctab_v, [kidx + 3 * KC_PAD])
        th_L = th0_v[pl.ds(s, 16)]
        th_nF = th1_v[pl.ds(s, 16)]
        th_G = th2_v[pl.ds(s, 16)]
        th_nS = th3_v[pl.ds(s, 16)]
        pT_v[pl.ds(s, 16)] = _sigmoid(pT_l + th_L)
        pF_v[pl.ds(s, 16)] = _sigmoid(pF_l - th_nF)
        xg_v[pl.ds(s, 16)] = pG_l + th_G
        xs_v[pl.ds(s, 16)] = pS_l - th_nS
        return carry

    for j in range(NCHUNK):
        for g in gathers[j]:
            g.wait()
        lax.fori_loop(j * SPC, (j + 1) * SPC, step, 0)

    outs = [
        pltpu.async_copy(pT_v, pT_hbm.at[pl.ds(base, BPW)], sem),
        pltpu.async_copy(pF_v, pF_hbm.at[pl.ds(base, BPW)], sem),
        pltpu.async_copy(xg_v, xg_hbm.at[pl.ds(base, BPW)], sem),
        pltpu.async_copy(xs_v, xs_hbm.at[pl.ds(base, BPW)], sem),
    ]
    for o in outs:
        o.wait()


def _update_body(omega_hbm, sigma_hbm, hp_hbm, obs_hbm,
                 pT_hbm, pF_hbm, xg_hbm, xs_hbm, pid_hbm,
                 hnew_hbm, pc_hbm,
                 hp_v, obs_v, pidx_v, pT_v, pF_v, xg_v, xs_v,
                 om_v, sg_v, hn_v, pc_v, sem, gsem):
    wid = lax.axis_index("s") * NC + lax.axis_index("c")
    base = wid * BPW
    base2 = wid * (2 * BPW)

    idx_copy = pltpu.async_copy(pid_hbm.at[pl.ds(base, BPW)], pidx_v, sem)
    lin_copies = [
        pltpu.async_copy(hp_hbm.at[pl.ds(base2, 2 * BPW)], hp_v, sem),
        pltpu.async_copy(obs_hbm.at[pl.ds(base, BPW)], obs_v, sem),
        pltpu.async_copy(pT_hbm.at[pl.ds(base, BPW)], pT_v, sem),
        pltpu.async_copy(pF_hbm.at[pl.ds(base, BPW)], pF_v, sem),
        pltpu.async_copy(xg_hbm.at[pl.ds(base, BPW)], xg_v, sem),
        pltpu.async_copy(xs_hbm.at[pl.ds(base, BPW)], xs_v, sem),
    ]
    idx_copy.wait()

    gathers = []
    for j in range(NCHUNK):
        sl = pl.ds(j * CHUNK, CHUNK)
        gathers.append([
            pltpu.async_copy(omega_hbm.at[pidx_v.at[sl]], om_v.at[sl],
                             gsem.at[j]),
            pltpu.async_copy(sigma_hbm.at[pidx_v.at[sl]], sg_v.at[sl],
                             gsem.at[j]),
        ])
    for c in lin_copies:
        c.wait()

    def step(i, carry):
        s = pl.multiple_of(i * 16, 16)
        su = pl.multiple_of((i // 8) * 256 + (i % 8) * 16, 16)

        om = om_v[pl.ds(s, 16)]
        sg = sg_v[pl.ds(s, 16)]
        pT = pT_v[pl.ds(s, 16)]
        pF = pF_v[pl.ds(s, 16)]
        pG = _sigmoid(xg_v[pl.ds(s, 16)] + om)
        pS = _sigmoid(xs_v[pl.ds(s, 16)] + sg)

        h_u = hp_v[pl.ds(su, 16)]
        h_m = hp_v[pl.ds(su + CHUNK, 16)]
        obs_b = obs_v[pl.ds(s, 16)] != 0

        p_m = jnp.where(obs_b, 1.0 - pS, pS)
        p_u = jnp.where(obs_b, pG, 1.0 - pG)

        a_u = p_u * h_u
        a_m = p_m * h_m
        new_m = (1.0 - pF) * a_m + pT * a_u
        new_u = pF * a_m + (1.0 - pT) * a_u
        inv_norm = 1.0 / (new_m + new_u + EPSILON)
        new_m = new_m * inv_norm
        new_u = new_u * inv_norm
        pc = (1.0 - pS) * new_m + pG * new_u

        hn_v[pl.ds(su, 16)] = new_u
        hn_v[pl.ds(su + CHUNK, 16)] = new_m
        pc_v[pl.ds(s, 16)] = pc
        return carry

    for j in range(NCHUNK):
        for g in gathers[j]:
            g.wait()
        lax.fori_loop(j * SPC, (j + 1) * SPC, step, 0)

    pltpu.async_copy(hn_v, hnew_hbm.at[pl.ds(base2, 2 * BPW)], sem).wait()
    pltpu.async_copy(pc_v, pc_hbm.at[pl.ds(base, BPW)], sem).wait()


@jax.jit
def _bkt_sc(kctab, omega1d, sigma1d, thblk, hp_blk, observation,
            kc_ids, problem_ids, student_ids):
    mesh = plsc.VectorSubcoreMesh(core_axis_name="c", subcore_axis_name="s",
                                  num_cores=NC, num_subcores=NS)
    params = pltpu.CompilerParams(needs_layout_passes=False,
                                  disable_bounds_checks=True)
    out1d = jax.ShapeDtypeStruct((B,), jnp.float32)

    logits_fn = pl.kernel(
        _logits_body,
        out_type=(out1d, out1d, out1d, out1d),
        mesh=mesh,
        scratch_types=[
            pltpu.VMEM((4 * KC_PAD,), jnp.float32),
            pltpu.VMEM((BPW,), jnp.int32),
            pltpu.VMEM((BPW,), jnp.int32),
            pltpu.VMEM((BPW,), jnp.int32),
            pltpu.VMEM((BPW,), jnp.int32),
            pltpu.VMEM((BPW,), jnp.int32),
            pltpu.VMEM((BPW,), jnp.int32),
            pltpu.VMEM((BPW,), jnp.float32),
            pltpu.VMEM((BPW,), jnp.float32),
            pltpu.VMEM((BPW,), jnp.float32),
            pltpu.VMEM((BPW,), jnp.float32),
            pltpu.VMEM((BPW,), jnp.float32),
            pltpu.VMEM((BPW,), jnp.float32),
            pltpu.VMEM((BPW,), jnp.float32),
            pltpu.VMEM((BPW,), jnp.float32),
            pltpu.SemaphoreType.DMA,
            pltpu.SemaphoreType.DMA((NCHUNK,)),
        ],
        compiler_params=params,
        name="bkt_logits",
    )
    pT_a, pF_a, xg_a, xs_a = logits_fn(kctab, thblk, kc_ids, student_ids)

    update_fn = pl.kernel(
        _update_body,
        out_type=(
            jax.ShapeDtypeStruct((2 * B,), jnp.float32),
            out1d,
        ),
        mesh=mesh,
        scratch_types=[
            pltpu.VMEM((2 * BPW,), jnp.float32),
            pltpu.VMEM((BPW,), jnp.int32),
            pltpu.VMEM((BPW,), jnp.int32),
            pltpu.VMEM((BPW,), jnp.float32),
            pltpu.VMEM((BPW,), jnp.float32),
            pltpu.VMEM((BPW,), jnp.float32),
            pltpu.VMEM((BPW,), jnp.float32),
            pltpu.VMEM((BPW,), jnp.float32),
            pltpu.VMEM((BPW,), jnp.float32),
            pltpu.VMEM((2 * BPW,), jnp.float32),
            pltpu.VMEM((BPW,), jnp.float32),
            pltpu.SemaphoreType.DMA,
            pltpu.SemaphoreType.DMA((NCHUNK,)),
        ],
        compiler_params=params,
        name="bkt_update",
    )
    return update_fn(omega1d, sigma1d, hp_blk, observation,
                     pT_a, pF_a, xg_a, xs_a, problem_ids)


def kernel(h_prev, observation, kc_ids, problem_ids, student_ids,
           pT_logit, pF_logit, pG_logit, pS_logit, omega, sigma,
           student_ability):
    sigma1d = sigma.reshape(-1)
    (sigma1d, student_ability, pT_logit, pF_logit, pG_logit,
     pS_logit) = jax.lax.optimization_barrier(
        (sigma1d, student_ability, pT_logit, pF_logit, pG_logit, pS_logit))
    pad = KC_PAD - NUM_KCS
    kctab = jnp.concatenate([
        jnp.pad(pT_logit, (0, pad)),
        jnp.pad(pF_logit, (0, pad)),
        jnp.pad(pG_logit, (0, pad)),
        jnp.pad(pS_logit, (0, pad)),
    ])
    omega1d = omega.reshape(-1)
    thblk = jnp.concatenate([
        student_ability[:TH_MAIN]
        .reshape(TH_MAIN // CHUNK, CHUNK, 4).transpose(0, 2, 1).reshape(-1),
        student_ability[TH_MAIN:].T.reshape(-1),
    ])
    hp_blk = h_prev.reshape(B // CHUNK, CHUNK, 2).transpose(0, 2, 1).reshape(-1)
    h_blk, p_correct = _bkt_sc(kctab, omega1d, sigma1d, thblk,
                               hp_blk, observation,
                               kc_ids, problem_ids, student_ids)
    h_new = h_blk.reshape(B // CHUNK, 2, CHUNK).transpose(0, 2, 1).reshape(B, 2)
    return (h_new, p_correct)
